# baseline (speedup 1.0000x reference)
import functools

import jax
import jax.numpy as jnp
from jax import lax
from jax.experimental import pallas as pl
from jax.experimental.pallas import tpu as pltpu
from jax.experimental.pallas import tpu_sc as plsc

N_NODES = 50000
N_EDGES = 800000
HIDDEN = 128
N_LAYERS = 3
NUM_GRAPHS = 8

R = 512
NPAD = 50176
GRID = NPAD // R

NCHUNK = 8
CH = NPAD // NCHUNK
TRASH = CH
ACC = CH + 16
K = 256
KROWS = K // 128
EROWS = N_EDGES // 128
STRIPE = CH // 16


def _ln(v, g, b):
    m = jnp.mean(v, axis=-1, keepdims=True)
    var = jnp.mean((v - m) ** 2, axis=-1, keepdims=True)
    return (v - m) / jnp.sqrt(var + 1e-5) * g + b


def _dot(a, b):
    return lax.dot_general(a, b, (((1,), (0,)), ((), ())),
                           preferred_element_type=jnp.float32)


def _pre_body(xo, xa, xg,
              aW1, ab1, ag, abn, aW2, ab2,
              oW1, ob1, og, obn, oW2, ob2,
              Wn0, Wn1, Wn2, bn, out):
    az = jax.nn.relu(_dot(xa[...], aW1[...]) + ab1[...])
    az = _ln(az, ag[...], abn[...])
    az = _dot(az, aW2[...]) + ab2[...]
    oz = jax.nn.relu(_dot(xg[...], oW1[...]) + ob1[...])
    oz = _ln(oz, og[...], obn[...])
    oz = _dot(oz, oW2[...]) + ob2[...]
    h = (_dot(xo[...], Wn0[...]) + _dot(az, Wn1[...])
         + _dot(oz, Wn2[...]) + bn[...])
    out[...] = h


def _stage_a(xo, xa, xg, p):
    full = lambda s: pl.BlockSpec(s, lambda i: (0, 0))
    return pl.pallas_call(
        _pre_body,
        grid=(GRID,),
        in_specs=[
            pl.BlockSpec((R, 16), lambda i: (i, 0)),
            pl.BlockSpec((R, 8), lambda i: (i, 0)),
            pl.BlockSpec((R, 8), lambda i: (i, 0)),
            full((8, 16)), full((1, 16)), full((1, 16)), full((1, 16)),
            full((16, 16)), full((1, 16)),
            full((8, 16)), full((1, 16)), full((1, 16)), full((1, 16)),
            full((16, 16)), full((1, 16)),
            full((16, HIDDEN)), full((16, HIDDEN)), full((16, HIDDEN)),
            full((1, HIDDEN)),
        ],
        out_specs=pl.BlockSpec((R, HIDDEN), lambda i: (i, 0)),
        out_shape=jax.ShapeDtypeStruct((NPAD, HIDDEN), jnp.float32),
    )(xo, xa, xg,
      _p2(p['axis_W1'], 8), _r2(p['axis_b1']), _r2(p['axis_g']),
      _r2(p['axis_bn']), p['axis_W2'], _r2(p['axis_b2']),
      _p2(p['origin_W1'], 8), _r2(p['origin_b1']), _r2(p['origin_g']),
      _r2(p['origin_bn']), p['origin_W2'], _r2(p['origin_b2']),
      _p2(p['node_in_W'][:13], 16), _p2(p['node_in_W'][13:29], 16),
      _p2(p['node_in_W'][29:45], 16), _r2(p['node_in_b']))


def _p2(w, rows):
    return jnp.pad(w, ((0, rows - w.shape[0]), (0, 0)))


def _r2(b):
    return b.reshape(1, -1)


def _layer_body(scale, h, agg, W1, b1, g, bn, W2, b2, ng, nb, out):
    m = scale[0, 0] * h[...] + agg[...]
    t = jax.nn.relu(_dot(m, W1[...]) + b1[...])
    t = _ln(t, g[...], bn[...])
    t = _dot(t, W2[...]) + b2[...]
    out[...] = _ln(jax.nn.relu(t), ng[...], nb[...]) + h[...]


def _layer(h, agg, p, i):
    pre = 'conv%d' % i
    scale = (1.0 + p['eps%d' % i]).reshape(1, 1)
    fullh = lambda: pl.BlockSpec((HIDDEN, HIDDEN), lambda j: (0, 0))
    row = lambda: pl.BlockSpec((1, HIDDEN), lambda j: (0, 0))
    return pl.pallas_call(
        _layer_body,
        grid=(GRID,),
        in_specs=[
            pl.BlockSpec(memory_space=pltpu.SMEM),
            pl.BlockSpec((R, HIDDEN), lambda j: (j, 0)),
            pl.BlockSpec((R, HIDDEN), lambda j: (j, 0)),
            fullh(), row(), row(), row(), fullh(), row(), row(), row(),
        ],
        out_specs=pl.BlockSpec((R, HIDDEN), lambda j: (j, 0)),
        out_shape=jax.ShapeDtypeStruct((NPAD, HIDDEN), jnp.float32),
    )(scale, h, agg,
      p[pre + '_W1'], _r2(p[pre + '_b1']), _r2(p[pre + '_g']),
      _r2(p[pre + '_bn']), p[pre + '_W2'], _r2(p[pre + '_b2']),
      _r2(p['norm%d_g' % i]), _r2(p['norm%d_b' % i]))


def _agg_body(h_hbm, word_hbm, tbl_hbm, zeros_hbm, agg_hbm,
              acc, tbuf, zbuf, wbuf, sbuf, dbuf, gbuf, sem):
    ci = lax.axis_index("c")
    si = lax.axis_index("s")
    lanes = lax.iota(jnp.int32, 16)
    pltpu.sync_copy(tbl_hbm, tbuf)
    pltpu.sync_copy(zeros_hbm, zbuf)
    ti = (ci * 16 + si) * 4
    for pp in range(NCHUNK // 2):
        chunk = ci * (NCHUNK // 2) + pp
        for k in range(4):
            pltpu.sync_copy(
                zbuf, acc.at[pl.ds(si * STRIPE + k * (STRIPE // 4),
                                   STRIPE // 4)])
        plsc.subcore_barrier()
        tv = tbuf[pl.ds(pp * 128 + ti, 16)]
        off0row = tv[0]
        ngroup = tv[1]
        elo = tv[2]
        ehi = tv[3]

        def body(qb, carry):
            row = pl.multiple_of(off0row + qb * 8, 8)
            pltpu.sync_copy(word_hbm.at[pl.ds(row, 8)], wbuf)
            base = row * 128
            for j in range(64):
                pos = base + j * 16 + lanes
                w = wbuf[j // 8, pl.ds((j % 8) * 16, 16)]
                ok = (pos >= elo) & (pos < ehi)
                sbuf[j // 8, pl.ds((j % 8) * 16, 16)] = (
                    (w >> 13) & jnp.int32(0xFFFF))
                dbuf[j // 8, pl.ds((j % 8) * 16, 16)] = jnp.where(
                    ok, w & jnp.int32(0x1FFF), jnp.int32(TRASH))
            cps = [pltpu.async_copy(h_hbm.at[sbuf.at[r]],
                                    gbuf.at[pl.ds(r * 128, 128)], sem)
                   for r in range(2)]
            for r in range(8):
                if r < 6:
                    cps.append(pltpu.async_copy(
                        h_hbm.at[sbuf.at[r + 2]],
                        gbuf.at[pl.ds(((r + 2) % 3) * 128, 128)], sem))
                cps[r].wait()
                pltpu.sync_copy(gbuf.at[pl.ds((r % 3) * 128, 128)],
                                acc.at[dbuf.at[r]], add=True)
            return carry

        lax.fori_loop(0, ngroup * 4, body, jnp.int32(0))
        plsc.subcore_barrier()
        rowbase = chunk * CH + si * STRIPE
        pltpu.sync_copy(acc.at[pl.ds(si * STRIPE, STRIPE)],
                        agg_hbm.at[pl.ds(rowbase, STRIPE)])
        plsc.subcore_barrier()


_agg_call = pl.kernel(
    _agg_body,
    out_type=jax.ShapeDtypeStruct((NPAD, HIDDEN), jnp.float32),
    mesh=plsc.VectorSubcoreMesh(core_axis_name="c", subcore_axis_name="s"),
    scratch_types=[
        pltpu.VMEM_SHARED((ACC, HIDDEN), jnp.float32),
        pltpu.VMEM((NCHUNK * 64 + 16,), jnp.int32),
        pltpu.VMEM((STRIPE // 4, HIDDEN), jnp.float32),
        pltpu.VMEM((8, 128), jnp.int32),
        pltpu.VMEM((8, 128), jnp.int32),
        pltpu.VMEM((8, 128), jnp.int32),
        pltpu.VMEM((384, HIDDEN), jnp.float32),
        pltpu.SemaphoreType.DMA,
    ],
)


def _edge_plan(edge_index):
    src = edge_index[0].astype(jnp.uint32)
    dst = edge_index[1].astype(jnp.uint32)
    word = ((dst // CH) << 29) | (src << 13) | (dst % CH)
    s_word = jnp.sort(word)
    wordrows = lax.bitcast_convert_type(
        jnp.pad(s_word, (0, 4096)), jnp.int32).reshape(EROWS + 32, 128)
    eb = jnp.concatenate([
        jnp.zeros((1,), jnp.int32),
        jnp.searchsorted(
            s_word, jnp.arange(1, NCHUNK, dtype=jnp.uint32) << 29
        ).astype(jnp.int32),
        jnp.full((1,), N_EDGES, jnp.int32),
    ])
    astart = (eb[:NCHUNK] // 1024) * 1024
    bc4 = (eb[1:] - astart + 1023) // 1024
    t = jnp.arange(16, dtype=jnp.int32)
    q0 = (t[None, :] * bc4[:, None]) // 16
    q1 = ((t[None, :] + 1) * bc4[:, None]) // 16
    off0row = astart[:, None] // 128 + q0 * 8
    ngroup = (q1 - q0 + 3) // 4
    elo = jnp.maximum(eb[:NCHUNK, None], astart[:, None] + q0 * 1024)
    ehi = jnp.minimum(eb[1:, None], astart[:, None] + q1 * 1024)
    tblc = jnp.stack([off0row, ngroup, elo, ehi], axis=1)
    half = NCHUNK // 2
    tbl = jnp.stack([tblc[jnp.array([pp, half + pp])] for pp in range(half)],
                    axis=0)
    tbl = jnp.pad(tbl.transpose(0, 1, 3, 2).reshape(NCHUNK * 64), (0, 16))
    zeros = jnp.zeros((STRIPE // 4, HIDDEN), jnp.float32)
    return wordrows, tbl, zeros


def _gate_body(h, oh, W1, b1, g, bn, W2, b2, gate, gmax):
    i = pl.program_id(0)
    t = jax.nn.relu(_dot(h[...], W1[...]) + b1[...])
    t = _ln(t, g[...], bn[...])
    gt = _dot(t, W2[...]) + b2[...]
    gate[...] = gt

    @pl.when(i == 0)
    def _():
        gmax[...] = jnp.full((1, NUM_GRAPHS), -1e30, jnp.float32)

    masked = jnp.where(oh[...] > 0, gt, -1e30)
    gmax[...] = jnp.maximum(gmax[...], jnp.max(masked, axis=0, keepdims=True))


def _gate(h, onehot, p):
    full = lambda s: pl.BlockSpec(s, lambda i: (0, 0))
    return pl.pallas_call(
        _gate_body,
        grid=(GRID,),
        in_specs=[
            pl.BlockSpec((R, HIDDEN), lambda i: (i, 0)),
            pl.BlockSpec((R, NUM_GRAPHS), lambda i: (i, 0)),
            full((HIDDEN, 64)), full((1, 64)), full((1, 64)), full((1, 64)),
            full((64, 1)), full((1, 1)),
        ],
        out_specs=[pl.BlockSpec((R, 1), lambda i: (i, 0)),
                   pl.BlockSpec((1, NUM_GRAPHS), lambda i: (0, 0))],
        out_shape=[jax.ShapeDtypeStruct((NPAD, 1), jnp.float32),
                   jax.ShapeDtypeStruct((1, NUM_GRAPHS), jnp.float32)],
    )(h, onehot, p['gate_W1'], _r2(p['gate_b1']), _r2(p['gate_g']),
      _r2(p['gate_bn']), p['gate_W2'], _r2(p['gate_b2']))


def _pool_body(h, gate, oh, gmax, den, gs):
    i = pl.program_id(0)
    gmaxrow = _dot(oh[...], gmax[...])
    e = jnp.exp(jnp.minimum(gate[...] - gmaxrow, 60.0))
    valid = jnp.sum(oh[...], axis=1, keepdims=True) > 0
    e = jnp.where(valid, e, 0.0)

    @pl.when(i == 0)
    def _():
        den[...] = jnp.zeros((NUM_GRAPHS, 1), jnp.float32)
        gs[...] = jnp.zeros((NUM_GRAPHS, HIDDEN), jnp.float32)

    ct = lambda a, b: lax.dot_general(a, b, (((0,), (0,)), ((), ())),
                                      preferred_element_type=jnp.float32)
    den[...] += ct(oh[...], e)
    gs[...] += ct(oh[...], e * h[...])


def _pool(h, gate, onehot, gmaxc):
    full = lambda s: pl.BlockSpec(s, lambda i: (0, 0))
    return pl.pallas_call(
        _pool_body,
        grid=(GRID,),
        in_specs=[
            pl.BlockSpec((R, HIDDEN), lambda i: (i, 0)),
            pl.BlockSpec((R, 1), lambda i: (i, 0)),
            pl.BlockSpec((R, NUM_GRAPHS), lambda i: (i, 0)),
            full((NUM_GRAPHS, 1)),
        ],
        out_specs=[pl.BlockSpec((NUM_GRAPHS, 1), lambda i: (0, 0)),
                   pl.BlockSpec((NUM_GRAPHS, HIDDEN), lambda i: (0, 0))],
        out_shape=[jax.ShapeDtypeStruct((NUM_GRAPHS, 1), jnp.float32),
                   jax.ShapeDtypeStruct((NUM_GRAPHS, HIDDEN), jnp.float32)],
    )(h, gate, onehot, gmaxc)


def _head_body(gs, den, W1, b1, g, bn, W2, b2, out):
    gv = gs[...] / (den[...] + 1e-16)
    t = jax.nn.relu(_dot(gv, W1[...]) + b1[...])
    t = _ln(t, g[...], bn[...])
    out[...] = _dot(t, W2[...]) + b2[...]


def _head(gs, den, p):
    full = lambda s: pl.BlockSpec(s, lambda i: (0, 0))
    return pl.pallas_call(
        _head_body,
        grid=(1,),
        in_specs=[
            full((NUM_GRAPHS, HIDDEN)), full((NUM_GRAPHS, 1)),
            full((HIDDEN, 64)), full((1, 64)), full((1, 64)), full((1, 64)),
            full((64, 1)), full((1, 1)),
        ],
        out_specs=full((NUM_GRAPHS, 1)),
        out_shape=jax.ShapeDtypeStruct((NUM_GRAPHS, 1), jnp.float32),
    )(gs, den, p['head_W1'], _r2(p['head_b1']), _r2(p['head_g']),
      _r2(p['head_bn']), p['head_W2'], _r2(p['head_b2']))


def kernel(x, params, edge_index, batch):
    p = params
    padn = lambda a: jnp.pad(a, ((0, NPAD - a.shape[0]), (0, 0)))
    x_oth = padn(jnp.pad(jnp.concatenate([x[:, :9], x[:, 15:19]], axis=1),
                         ((0, 0), (0, 3))))
    x_axis = padn(jnp.pad(x[:, 9:12], ((0, 0), (0, 5))))
    x_orig = padn(jnp.pad(x[:, 12:15], ((0, 0), (0, 5))))

    h = _stage_a(x_oth, x_axis, x_orig, p)

    wordrows, tbl, zeros = _edge_plan(edge_index)
    for i in range(N_LAYERS):
        agg = _agg_call(h, wordrows, tbl, zeros)
        h = _layer(h, agg, p, i)

    bpad = jnp.pad(batch, (0, NPAD - batch.shape[0]),
                   constant_values=NUM_GRAPHS).astype(jnp.int32)
    onehot = (bpad[:, None] == jnp.arange(NUM_GRAPHS, dtype=jnp.int32)[None, :]
              ).astype(jnp.float32)
    gate, gmax = _gate(h, onehot, p)
    den, gs = _pool(h, gate, onehot, gmax.reshape(NUM_GRAPHS, 1))
    logit = _head(gs, den, p)
    return logit.reshape(NUM_GRAPHS)


# speedup vs baseline: 1.2328x; 1.2328x over previous
import functools

import jax
import jax.numpy as jnp
from jax import lax
from jax.experimental import pallas as pl
from jax.experimental.pallas import tpu as pltpu
from jax.experimental.pallas import tpu_sc as plsc

N_NODES = 50000
N_EDGES = 800000
HIDDEN = 128
N_LAYERS = 3
NUM_GRAPHS = 8

R = 512
NPAD = 50176
GRID = NPAD // R

NCHUNK = 8
CH = NPAD // NCHUNK
TRASH = CH
ACC = CH + 16
K = 256
KROWS = K // 128
EROWS = N_EDGES // 128
STRIPE = CH // 16


def _ln(v, g, b):
    m = jnp.mean(v, axis=-1, keepdims=True)
    var = jnp.mean((v - m) ** 2, axis=-1, keepdims=True)
    return (v - m) / jnp.sqrt(var + 1e-5) * g + b


def _dot(a, b):
    return lax.dot_general(a, b, (((1,), (0,)), ((), ())),
                           preferred_element_type=jnp.float32)


def _pre_body(xo, xa, xg,
              aW1, ab1, ag, abn, aW2, ab2,
              oW1, ob1, og, obn, oW2, ob2,
              Wn0, Wn1, Wn2, bn, out):
    az = jax.nn.relu(_dot(xa[...], aW1[...]) + ab1[...])
    az = _ln(az, ag[...], abn[...])
    az = _dot(az, aW2[...]) + ab2[...]
    oz = jax.nn.relu(_dot(xg[...], oW1[...]) + ob1[...])
    oz = _ln(oz, og[...], obn[...])
    oz = _dot(oz, oW2[...]) + ob2[...]
    h = (_dot(xo[...], Wn0[...]) + _dot(az, Wn1[...])
         + _dot(oz, Wn2[...]) + bn[...])
    out[...] = h


def _stage_a(xo, xa, xg, p):
    full = lambda s: pl.BlockSpec(s, lambda i: (0, 0))
    return pl.pallas_call(
        _pre_body,
        grid=(GRID,),
        in_specs=[
            pl.BlockSpec((R, 16), lambda i: (i, 0)),
            pl.BlockSpec((R, 8), lambda i: (i, 0)),
            pl.BlockSpec((R, 8), lambda i: (i, 0)),
            full((8, 16)), full((1, 16)), full((1, 16)), full((1, 16)),
            full((16, 16)), full((1, 16)),
            full((8, 16)), full((1, 16)), full((1, 16)), full((1, 16)),
            full((16, 16)), full((1, 16)),
            full((16, HIDDEN)), full((16, HIDDEN)), full((16, HIDDEN)),
            full((1, HIDDEN)),
        ],
        out_specs=pl.BlockSpec((R, HIDDEN), lambda i: (i, 0)),
        out_shape=jax.ShapeDtypeStruct((NPAD, HIDDEN), jnp.float32),
    )(xo, xa, xg,
      _p2(p['axis_W1'], 8), _r2(p['axis_b1']), _r2(p['axis_g']),
      _r2(p['axis_bn']), p['axis_W2'], _r2(p['axis_b2']),
      _p2(p['origin_W1'], 8), _r2(p['origin_b1']), _r2(p['origin_g']),
      _r2(p['origin_bn']), p['origin_W2'], _r2(p['origin_b2']),
      _p2(p['node_in_W'][:13], 16), _p2(p['node_in_W'][13:29], 16),
      _p2(p['node_in_W'][29:45], 16), _r2(p['node_in_b']))


def _p2(w, rows):
    return jnp.pad(w, ((0, rows - w.shape[0]), (0, 0)))


def _r2(b):
    return b.reshape(1, -1)


def _layer_body(scale, h, agg, W1, b1, g, bn, W2, b2, ng, nb, out):
    m = scale[0, 0] * h[...] + agg[...]
    t = jax.nn.relu(_dot(m, W1[...]) + b1[...])
    t = _ln(t, g[...], bn[...])
    t = _dot(t, W2[...]) + b2[...]
    out[...] = _ln(jax.nn.relu(t), ng[...], nb[...]) + h[...]


def _layer(h, agg, p, i):
    pre = 'conv%d' % i
    scale = (1.0 + p['eps%d' % i]).reshape(1, 1)
    fullh = lambda: pl.BlockSpec((HIDDEN, HIDDEN), lambda j: (0, 0))
    row = lambda: pl.BlockSpec((1, HIDDEN), lambda j: (0, 0))
    return pl.pallas_call(
        _layer_body,
        grid=(GRID,),
        in_specs=[
            pl.BlockSpec(memory_space=pltpu.SMEM),
            pl.BlockSpec((R, HIDDEN), lambda j: (j, 0)),
            pl.BlockSpec((R, HIDDEN), lambda j: (j, 0)),
            fullh(), row(), row(), row(), fullh(), row(), row(), row(),
        ],
        out_specs=pl.BlockSpec((R, HIDDEN), lambda j: (j, 0)),
        out_shape=jax.ShapeDtypeStruct((NPAD, HIDDEN), jnp.float32),
    )(scale, h, agg,
      p[pre + '_W1'], _r2(p[pre + '_b1']), _r2(p[pre + '_g']),
      _r2(p[pre + '_bn']), p[pre + '_W2'], _r2(p[pre + '_b2']),
      _r2(p['norm%d_g' % i]), _r2(p['norm%d_b' % i]))


def _agg_body(h_hbm, word_hbm, tbl_hbm, zeros_hbm, agg_hbm,
              acc, tbuf, zbuf, wbuf, sbuf, dbuf, gbuf, sem):
    ci = lax.axis_index("c")
    si = lax.axis_index("s")
    lanes = lax.iota(jnp.int32, 16)
    pltpu.sync_copy(tbl_hbm, tbuf)
    pltpu.sync_copy(zeros_hbm, zbuf)
    ti = (ci * 16 + si) * 4
    for pp in range(NCHUNK // 2):
        chunk = ci * (NCHUNK // 2) + pp
        for k in range(4):
            pltpu.sync_copy(
                zbuf, acc.at[pl.ds(si * STRIPE + k * (STRIPE // 4),
                                   STRIPE // 4)])
        plsc.subcore_barrier()
        tv = tbuf[pl.ds(pp * 128 + ti, 16)]
        off0row = tv[0]
        nquad = tv[1]
        elo = tv[2]
        ehi = tv[3]

        def body(qb, carry):
            row = pl.multiple_of(off0row + qb * 8, 8)
            pltpu.sync_copy(word_hbm.at[pl.ds(row, 8)], wbuf)
            base = row * 128
            for j in range(64):
                pos = base + j * 16 + lanes
                w = wbuf[j // 8, pl.ds((j % 8) * 16, 16)]
                ok = (pos >= elo) & (pos < ehi)
                sbuf[j // 8, pl.ds((j % 8) * 16, 16)] = (
                    (w >> 13) & jnp.int32(0xFFFF))
                dbuf[j // 8, pl.ds((j % 8) * 16, 16)] = jnp.where(
                    ok, w & jnp.int32(0x1FFF), jnp.int32(TRASH))
            cps = [pltpu.async_copy(h_hbm.at[sbuf.at[r]],
                                    gbuf.at[pl.ds(r * 128, 128)], sem)
                   for r in range(2)]
            for r in range(8):
                if r < 6:
                    cps.append(pltpu.async_copy(
                        h_hbm.at[sbuf.at[r + 2]],
                        gbuf.at[pl.ds(((r + 2) % 3) * 128, 128)], sem))
                cps[r].wait()
                pltpu.sync_copy(gbuf.at[pl.ds((r % 3) * 128, 128)],
                                acc.at[dbuf.at[r]], add=True)
            return carry

        lax.fori_loop(0, nquad, body, jnp.int32(0))
        plsc.subcore_barrier()
        rowbase = chunk * CH + si * STRIPE
        pltpu.sync_copy(acc.at[pl.ds(si * STRIPE, STRIPE)],
                        agg_hbm.at[pl.ds(rowbase, STRIPE)])
        plsc.subcore_barrier()


_agg_call = pl.kernel(
    _agg_body,
    out_type=jax.ShapeDtypeStruct((NPAD, HIDDEN), jnp.float32),
    mesh=plsc.VectorSubcoreMesh(core_axis_name="c", subcore_axis_name="s"),
    scratch_types=[
        pltpu.VMEM_SHARED((ACC, HIDDEN), jnp.float32),
        pltpu.VMEM((NCHUNK * 64 + 16,), jnp.int32),
        pltpu.VMEM((STRIPE // 4, HIDDEN), jnp.float32),
        pltpu.VMEM((8, 128), jnp.int32),
        pltpu.VMEM((8, 128), jnp.int32),
        pltpu.VMEM((8, 128), jnp.int32),
        pltpu.VMEM((384, HIDDEN), jnp.float32),
        pltpu.SemaphoreType.DMA,
    ],
)


def _edge_plan(edge_index):
    src = edge_index[0].astype(jnp.uint32)
    dst = edge_index[1].astype(jnp.uint32)
    word = ((dst // CH) << 29) | (src << 13) | (dst % CH)
    s_word = jnp.sort(word)
    wordrows = lax.bitcast_convert_type(
        jnp.pad(s_word, (0, 4096)), jnp.int32).reshape(EROWS + 32, 128)
    eb = jnp.concatenate([
        jnp.zeros((1,), jnp.int32),
        jnp.searchsorted(
            s_word, jnp.arange(1, NCHUNK, dtype=jnp.uint32) << 29
        ).astype(jnp.int32),
        jnp.full((1,), N_EDGES, jnp.int32),
    ])
    astart = (eb[:NCHUNK] // 1024) * 1024
    bc4 = (eb[1:] - astart + 1023) // 1024
    t = jnp.arange(16, dtype=jnp.int32)
    q0 = (t[None, :] * bc4[:, None]) // 16
    q1 = ((t[None, :] + 1) * bc4[:, None]) // 16
    off0row = astart[:, None] // 128 + q0 * 8
    nquad = q1 - q0
    elo = jnp.maximum(eb[:NCHUNK, None], astart[:, None] + q0 * 1024)
    ehi = jnp.minimum(eb[1:, None], astart[:, None] + q1 * 1024)
    tblc = jnp.stack([off0row, nquad, elo, ehi], axis=1)
    half = NCHUNK // 2
    tbl = jnp.stack([tblc[jnp.array([pp, half + pp])] for pp in range(half)],
                    axis=0)
    tbl = jnp.pad(tbl.transpose(0, 1, 3, 2).reshape(NCHUNK * 64), (0, 16))
    zeros = jnp.zeros((STRIPE // 4, HIDDEN), jnp.float32)
    return wordrows, tbl, zeros


def _gate_body(h, oh, W1, b1, g, bn, W2, b2, gate, gmax):
    i = pl.program_id(0)
    t = jax.nn.relu(_dot(h[...], W1[...]) + b1[...])
    t = _ln(t, g[...], bn[...])
    gt = _dot(t, W2[...]) + b2[...]
    gate[...] = gt

    @pl.when(i == 0)
    def _():
        gmax[...] = jnp.full((1, NUM_GRAPHS), -1e30, jnp.float32)

    masked = jnp.where(oh[...] > 0, gt, -1e30)
    gmax[...] = jnp.maximum(gmax[...], jnp.max(masked, axis=0, keepdims=True))


def _gate(h, onehot, p):
    full = lambda s: pl.BlockSpec(s, lambda i: (0, 0))
    return pl.pallas_call(
        _gate_body,
        grid=(GRID,),
        in_specs=[
            pl.BlockSpec((R, HIDDEN), lambda i: (i, 0)),
            pl.BlockSpec((R, NUM_GRAPHS), lambda i: (i, 0)),
            full((HIDDEN, 64)), full((1, 64)), full((1, 64)), full((1, 64)),
            full((64, 1)), full((1, 1)),
        ],
        out_specs=[pl.BlockSpec((R, 1), lambda i: (i, 0)),
                   pl.BlockSpec((1, NUM_GRAPHS), lambda i: (0, 0))],
        out_shape=[jax.ShapeDtypeStruct((NPAD, 1), jnp.float32),
                   jax.ShapeDtypeStruct((1, NUM_GRAPHS), jnp.float32)],
    )(h, onehot, p['gate_W1'], _r2(p['gate_b1']), _r2(p['gate_g']),
      _r2(p['gate_bn']), p['gate_W2'], _r2(p['gate_b2']))


def _pool_body(h, gate, oh, gmax, den, gs):
    i = pl.program_id(0)
    gmaxrow = _dot(oh[...], gmax[...])
    e = jnp.exp(jnp.minimum(gate[...] - gmaxrow, 60.0))
    valid = jnp.sum(oh[...], axis=1, keepdims=True) > 0
    e = jnp.where(valid, e, 0.0)

    @pl.when(i == 0)
    def _():
        den[...] = jnp.zeros((NUM_GRAPHS, 1), jnp.float32)
        gs[...] = jnp.zeros((NUM_GRAPHS, HIDDEN), jnp.float32)

    ct = lambda a, b: lax.dot_general(a, b, (((0,), (0,)), ((), ())),
                                      preferred_element_type=jnp.float32)
    den[...] += ct(oh[...], e)
    gs[...] += ct(oh[...], e * h[...])


def _pool(h, gate, onehot, gmaxc):
    full = lambda s: pl.BlockSpec(s, lambda i: (0, 0))
    return pl.pallas_call(
        _pool_body,
        grid=(GRID,),
        in_specs=[
            pl.BlockSpec((R, HIDDEN), lambda i: (i, 0)),
            pl.BlockSpec((R, 1), lambda i: (i, 0)),
            pl.BlockSpec((R, NUM_GRAPHS), lambda i: (i, 0)),
            full((NUM_GRAPHS, 1)),
        ],
        out_specs=[pl.BlockSpec((NUM_GRAPHS, 1), lambda i: (0, 0)),
                   pl.BlockSpec((NUM_GRAPHS, HIDDEN), lambda i: (0, 0))],
        out_shape=[jax.ShapeDtypeStruct((NUM_GRAPHS, 1), jnp.float32),
                   jax.ShapeDtypeStruct((NUM_GRAPHS, HIDDEN), jnp.float32)],
    )(h, gate, onehot, gmaxc)


def _head_body(gs, den, W1, b1, g, bn, W2, b2, out):
    gv = gs[...] / (den[...] + 1e-16)
    t = jax.nn.relu(_dot(gv, W1[...]) + b1[...])
    t = _ln(t, g[...], bn[...])
    out[...] = _dot(t, W2[...]) + b2[...]


def _head(gs, den, p):
    full = lambda s: pl.BlockSpec(s, lambda i: (0, 0))
    return pl.pallas_call(
        _head_body,
        grid=(1,),
        in_specs=[
            full((NUM_GRAPHS, HIDDEN)), full((NUM_GRAPHS, 1)),
            full((HIDDEN, 64)), full((1, 64)), full((1, 64)), full((1, 64)),
            full((64, 1)), full((1, 1)),
        ],
        out_specs=full((NUM_GRAPHS, 1)),
        out_shape=jax.ShapeDtypeStruct((NUM_GRAPHS, 1), jnp.float32),
    )(gs, den, p['head_W1'], _r2(p['head_b1']), _r2(p['head_g']),
      _r2(p['head_bn']), p['head_W2'], _r2(p['head_b2']))


def kernel(x, params, edge_index, batch):
    p = params
    padn = lambda a: jnp.pad(a, ((0, NPAD - a.shape[0]), (0, 0)))
    x_oth = padn(jnp.pad(jnp.concatenate([x[:, :9], x[:, 15:19]], axis=1),
                         ((0, 0), (0, 3))))
    x_axis = padn(jnp.pad(x[:, 9:12], ((0, 0), (0, 5))))
    x_orig = padn(jnp.pad(x[:, 12:15], ((0, 0), (0, 5))))

    h = _stage_a(x_oth, x_axis, x_orig, p)

    wordrows, tbl, zeros = _edge_plan(edge_index)
    for i in range(N_LAYERS):
        agg = _agg_call(h, wordrows, tbl, zeros)
        h = _layer(h, agg, p, i)

    bpad = jnp.pad(batch, (0, NPAD - batch.shape[0]),
                   constant_values=NUM_GRAPHS).astype(jnp.int32)
    onehot = (bpad[:, None] == jnp.arange(NUM_GRAPHS, dtype=jnp.int32)[None, :]
              ).astype(jnp.float32)
    gate, gmax = _gate(h, onehot, p)
    den, gs = _pool(h, gate, onehot, gmax.reshape(NUM_GRAPHS, 1))
    logit = _head(gs, den, p)
    return logit.reshape(NUM_GRAPHS)


# speedup vs baseline: 1.2536x; 1.0169x over previous
import functools

import jax
import jax.numpy as jnp
from jax import lax
from jax.experimental import pallas as pl
from jax.experimental.pallas import tpu as pltpu
from jax.experimental.pallas import tpu_sc as plsc

N_NODES = 50000
N_EDGES = 800000
HIDDEN = 128
N_LAYERS = 3
NUM_GRAPHS = 8

R = 512
NPAD = 50176
GRID = NPAD // R

NCHUNK = 8
CH = NPAD // NCHUNK
TRASH = CH
ACC = CH + 16
K = 256
KROWS = K // 128
EROWS = N_EDGES // 128
STRIPE = CH // 16


def _ln(v, g, b):
    m = jnp.mean(v, axis=-1, keepdims=True)
    var = jnp.mean((v - m) ** 2, axis=-1, keepdims=True)
    return (v - m) / jnp.sqrt(var + 1e-5) * g + b


def _dot(a, b):
    return lax.dot_general(a, b, (((1,), (0,)), ((), ())),
                           preferred_element_type=jnp.float32)


def _pre_body(xo, xa, xg,
              aW1, ab1, ag, abn, aW2, ab2,
              oW1, ob1, og, obn, oW2, ob2,
              Wn0, Wn1, Wn2, bn, out):
    az = jax.nn.relu(_dot(xa[...], aW1[...]) + ab1[...])
    az = _ln(az, ag[...], abn[...])
    az = _dot(az, aW2[...]) + ab2[...]
    oz = jax.nn.relu(_dot(xg[...], oW1[...]) + ob1[...])
    oz = _ln(oz, og[...], obn[...])
    oz = _dot(oz, oW2[...]) + ob2[...]
    h = (_dot(xo[...], Wn0[...]) + _dot(az, Wn1[...])
         + _dot(oz, Wn2[...]) + bn[...])
    out[...] = h


def _stage_a(xo, xa, xg, p):
    full = lambda s: pl.BlockSpec(s, lambda i: (0, 0))
    return pl.pallas_call(
        _pre_body,
        grid=(GRID,),
        in_specs=[
            pl.BlockSpec((R, 16), lambda i: (i, 0)),
            pl.BlockSpec((R, 8), lambda i: (i, 0)),
            pl.BlockSpec((R, 8), lambda i: (i, 0)),
            full((8, 16)), full((1, 16)), full((1, 16)), full((1, 16)),
            full((16, 16)), full((1, 16)),
            full((8, 16)), full((1, 16)), full((1, 16)), full((1, 16)),
            full((16, 16)), full((1, 16)),
            full((16, HIDDEN)), full((16, HIDDEN)), full((16, HIDDEN)),
            full((1, HIDDEN)),
        ],
        out_specs=pl.BlockSpec((R, HIDDEN), lambda i: (i, 0)),
        out_shape=jax.ShapeDtypeStruct((NPAD, HIDDEN), jnp.float32),
    )(xo, xa, xg,
      _p2(p['axis_W1'], 8), _r2(p['axis_b1']), _r2(p['axis_g']),
      _r2(p['axis_bn']), p['axis_W2'], _r2(p['axis_b2']),
      _p2(p['origin_W1'], 8), _r2(p['origin_b1']), _r2(p['origin_g']),
      _r2(p['origin_bn']), p['origin_W2'], _r2(p['origin_b2']),
      _p2(p['node_in_W'][:13], 16), _p2(p['node_in_W'][13:29], 16),
      _p2(p['node_in_W'][29:45], 16), _r2(p['node_in_b']))


def _p2(w, rows):
    return jnp.pad(w, ((0, rows - w.shape[0]), (0, 0)))


def _r2(b):
    return b.reshape(1, -1)


def _layer_body(scale, h, agg, W1, b1, g, bn, W2, b2, ng, nb, out):
    m = scale[0, 0] * h[...] + agg[...]
    t = jax.nn.relu(_dot(m, W1[...]) + b1[...])
    t = _ln(t, g[...], bn[...])
    t = _dot(t, W2[...]) + b2[...]
    out[...] = _ln(jax.nn.relu(t), ng[...], nb[...]) + h[...]


def _layer(h, agg, p, i):
    pre = 'conv%d' % i
    scale = (1.0 + p['eps%d' % i]).reshape(1, 1)
    fullh = lambda: pl.BlockSpec((HIDDEN, HIDDEN), lambda j: (0, 0))
    row = lambda: pl.BlockSpec((1, HIDDEN), lambda j: (0, 0))
    return pl.pallas_call(
        _layer_body,
        grid=(GRID,),
        in_specs=[
            pl.BlockSpec(memory_space=pltpu.SMEM),
            pl.BlockSpec((R, HIDDEN), lambda j: (j, 0)),
            pl.BlockSpec((R, HIDDEN), lambda j: (j, 0)),
            fullh(), row(), row(), row(), fullh(), row(), row(), row(),
        ],
        out_specs=pl.BlockSpec((R, HIDDEN), lambda j: (j, 0)),
        out_shape=jax.ShapeDtypeStruct((NPAD, HIDDEN), jnp.float32),
    )(scale, h, agg,
      p[pre + '_W1'], _r2(p[pre + '_b1']), _r2(p[pre + '_g']),
      _r2(p[pre + '_bn']), p[pre + '_W2'], _r2(p[pre + '_b2']),
      _r2(p['norm%d_g' % i]), _r2(p['norm%d_b' % i]))


def _agg_body(h_hbm, word_hbm, tbl_hbm, zeros_hbm, agg_hbm,
              acc, tbuf, zbuf, wbuf, sbuf, dbuf, gbuf, sem):
    ci = lax.axis_index("c")
    si = lax.axis_index("s")
    lanes = lax.iota(jnp.int32, 16)
    pltpu.sync_copy(tbl_hbm, tbuf)
    pltpu.sync_copy(zeros_hbm, zbuf)
    ti = (ci * 16 + si) * 4
    for pp in range(NCHUNK // 2):
        chunk = ci * (NCHUNK // 2) + pp
        for k in range(4):
            pltpu.sync_copy(
                zbuf, acc.at[pl.ds(si * STRIPE + k * (STRIPE // 4),
                                   STRIPE // 4)])
        plsc.subcore_barrier()
        tv = tbuf[pl.ds(pp * 128 + ti, 16)]
        off0row = tv[0]
        nquad = tv[1]
        elo = tv[2]
        ehi = tv[3]

        def body(qb, carry):
            row = pl.multiple_of(off0row + qb * 8, 8)
            pltpu.sync_copy(word_hbm.at[pl.ds(row, 8)], wbuf)
            base = row * 128
            for j in range(64):
                pos = base + j * 16 + lanes
                w = wbuf[j // 8, pl.ds((j % 8) * 16, 16)]
                ok = (pos >= elo) & (pos < ehi)
                sbuf[j // 8, pl.ds((j % 8) * 16, 16)] = (
                    (w >> 13) & jnp.int32(0xFFFF))
                dbuf[j // 8, pl.ds((j % 8) * 16, 16)] = jnp.where(
                    ok, w & jnp.int32(0x1FFF), jnp.int32(TRASH))
            cps = [pltpu.async_copy(h_hbm.at[sbuf.at[r]],
                                    gbuf.at[pl.ds(r * 128, 128)], sem)
                   for r in range(2)]
            for r in range(8):
                if r < 6:
                    cps.append(pltpu.async_copy(
                        h_hbm.at[sbuf.at[r + 2]],
                        gbuf.at[pl.ds(((r + 2) % 3) * 128, 128)], sem))
                cps[r].wait()
                pltpu.sync_copy(gbuf.at[pl.ds((r % 3) * 128, 128)],
                                acc.at[dbuf.at[r]], add=True)
            return carry

        lax.fori_loop(0, nquad, body, jnp.int32(0))
        plsc.subcore_barrier()
        rowbase = chunk * CH + si * STRIPE
        pltpu.sync_copy(acc.at[pl.ds(si * STRIPE, STRIPE)],
                        agg_hbm.at[pl.ds(rowbase, STRIPE)])
        plsc.subcore_barrier()


_agg_call = pl.kernel(
    _agg_body,
    out_type=jax.ShapeDtypeStruct((NPAD, HIDDEN), jnp.float32),
    mesh=plsc.VectorSubcoreMesh(core_axis_name="c", subcore_axis_name="s"),
    scratch_types=[
        pltpu.VMEM_SHARED((ACC, HIDDEN), jnp.float32),
        pltpu.VMEM((NCHUNK * 64 + 16,), jnp.int32),
        pltpu.VMEM((STRIPE // 4, HIDDEN), jnp.float32),
        pltpu.VMEM((8, 128), jnp.int32),
        pltpu.VMEM((8, 128), jnp.int32),
        pltpu.VMEM((8, 128), jnp.int32),
        pltpu.VMEM((384, HIDDEN), jnp.float32),
        pltpu.SemaphoreType.DMA,
    ],
)


def _edge_plan(edge_index):
    src = edge_index[0].astype(jnp.uint32)
    dst = edge_index[1].astype(jnp.uint32)
    word = ((dst // CH) << 29) | (src << 13) | (dst % CH)
    s_word = jnp.sort(word)
    wordrows = lax.bitcast_convert_type(
        jnp.pad(s_word, (0, 4096)), jnp.int32).reshape(EROWS + 32, 128)
    eb = jnp.concatenate([
        jnp.zeros((1,), jnp.int32),
        jnp.searchsorted(
            s_word, jnp.arange(1, NCHUNK, dtype=jnp.uint32) << 29
        ).astype(jnp.int32),
        jnp.full((1,), N_EDGES, jnp.int32),
    ])
    astart = (eb[:NCHUNK] // 1024) * 1024
    bc4 = (eb[1:] - astart + 1023) // 1024
    t = jnp.arange(16, dtype=jnp.int32)
    q0 = (t[None, :] * bc4[:, None]) // 16
    q1 = ((t[None, :] + 1) * bc4[:, None]) // 16
    off0row = astart[:, None] // 128 + q0 * 8
    nquad = q1 - q0
    elo = jnp.maximum(eb[:NCHUNK, None], astart[:, None] + q0 * 1024)
    ehi = jnp.minimum(eb[1:, None], astart[:, None] + q1 * 1024)
    tblc = jnp.stack([off0row, nquad, elo, ehi], axis=1)
    half = NCHUNK // 2
    tbl = jnp.stack([tblc[jnp.array([pp, half + pp])] for pp in range(half)],
                    axis=0)
    tbl = jnp.pad(tbl.transpose(0, 1, 3, 2).reshape(NCHUNK * 64), (0, 16))
    zeros = jnp.zeros((STRIPE // 4, HIDDEN), jnp.float32)
    return wordrows, tbl, zeros


def _layer_gate_body(scale, h, agg, W1, b1, g, bn, W2, b2, ng, nb,
                     oh, gW1, gb1, gg, gbn, gW2, gb2, out, gate, gmax):
    i = pl.program_id(0)
    m = scale[0, 0] * h[...] + agg[...]
    t = jax.nn.relu(_dot(m, W1[...]) + b1[...])
    t = _ln(t, g[...], bn[...])
    t = _dot(t, W2[...]) + b2[...]
    hn = _ln(jax.nn.relu(t), ng[...], nb[...]) + h[...]
    out[...] = hn
    t2 = jax.nn.relu(_dot(hn, gW1[...]) + gb1[...])
    t2 = _ln(t2, gg[...], gbn[...])
    gt = _dot(t2, gW2[...]) + gb2[...]
    gate[...] = gt

    @pl.when(i == 0)
    def _():
        gmax[...] = jnp.full((1, NUM_GRAPHS), -1e30, jnp.float32)

    masked = jnp.where(oh[...] > 0, gt, -1e30)
    gmax[...] = jnp.maximum(gmax[...], jnp.max(masked, axis=0, keepdims=True))


def _layer_gate(h, agg, onehot, p, i):
    pre = 'conv%d' % i
    scale = (1.0 + p['eps%d' % i]).reshape(1, 1)
    fullh = lambda: pl.BlockSpec((HIDDEN, HIDDEN), lambda j: (0, 0))
    row = lambda: pl.BlockSpec((1, HIDDEN), lambda j: (0, 0))
    full = lambda sh: pl.BlockSpec(sh, lambda j: (0, 0))
    return pl.pallas_call(
        _layer_gate_body,
        grid=(GRID,),
        in_specs=[
            pl.BlockSpec(memory_space=pltpu.SMEM),
            pl.BlockSpec((R, HIDDEN), lambda j: (j, 0)),
            pl.BlockSpec((R, HIDDEN), lambda j: (j, 0)),
            fullh(), row(), row(), row(), fullh(), row(), row(), row(),
            pl.BlockSpec((R, NUM_GRAPHS), lambda j: (j, 0)),
            full((HIDDEN, 64)), full((1, 64)), full((1, 64)), full((1, 64)),
            full((64, 1)), full((1, 1)),
        ],
        out_specs=[pl.BlockSpec((R, HIDDEN), lambda j: (j, 0)),
                   pl.BlockSpec((R, 1), lambda j: (j, 0)),
                   pl.BlockSpec((1, NUM_GRAPHS), lambda j: (0, 0))],
        out_shape=[jax.ShapeDtypeStruct((NPAD, HIDDEN), jnp.float32),
                   jax.ShapeDtypeStruct((NPAD, 1), jnp.float32),
                   jax.ShapeDtypeStruct((1, NUM_GRAPHS), jnp.float32)],
    )(scale, h, agg,
      p[pre + '_W1'], _r2(p[pre + '_b1']), _r2(p[pre + '_g']),
      _r2(p[pre + '_bn']), p[pre + '_W2'], _r2(p[pre + '_b2']),
      _r2(p['norm%d_g' % i]), _r2(p['norm%d_b' % i]),
      onehot, p['gate_W1'], _r2(p['gate_b1']), _r2(p['gate_g']),
      _r2(p['gate_bn']), p['gate_W2'], _r2(p['gate_b2']))


def _gate_body(h, oh, W1, b1, g, bn, W2, b2, gate, gmax):
    i = pl.program_id(0)
    t = jax.nn.relu(_dot(h[...], W1[...]) + b1[...])
    t = _ln(t, g[...], bn[...])
    gt = _dot(t, W2[...]) + b2[...]
    gate[...] = gt

    @pl.when(i == 0)
    def _():
        gmax[...] = jnp.full((1, NUM_GRAPHS), -1e30, jnp.float32)

    masked = jnp.where(oh[...] > 0, gt, -1e30)
    gmax[...] = jnp.maximum(gmax[...], jnp.max(masked, axis=0, keepdims=True))


def _gate(h, onehot, p):
    full = lambda s: pl.BlockSpec(s, lambda i: (0, 0))
    return pl.pallas_call(
        _gate_body,
        grid=(GRID,),
        in_specs=[
            pl.BlockSpec((R, HIDDEN), lambda i: (i, 0)),
            pl.BlockSpec((R, NUM_GRAPHS), lambda i: (i, 0)),
            full((HIDDEN, 64)), full((1, 64)), full((1, 64)), full((1, 64)),
            full((64, 1)), full((1, 1)),
        ],
        out_specs=[pl.BlockSpec((R, 1), lambda i: (i, 0)),
                   pl.BlockSpec((1, NUM_GRAPHS), lambda i: (0, 0))],
        out_shape=[jax.ShapeDtypeStruct((NPAD, 1), jnp.float32),
                   jax.ShapeDtypeStruct((1, NUM_GRAPHS), jnp.float32)],
    )(h, onehot, p['gate_W1'], _r2(p['gate_b1']), _r2(p['gate_g']),
      _r2(p['gate_bn']), p['gate_W2'], _r2(p['gate_b2']))


def _pool_body(h, gate, oh, gmax, hW1, hb1, hg, hbn, hW2, hb2,
               den, gs, logit):
    i = pl.program_id(0)
    gmaxrow = _dot(oh[...], gmax[...])
    e = jnp.exp(jnp.minimum(gate[...] - gmaxrow, 60.0))
    valid = jnp.sum(oh[...], axis=1, keepdims=True) > 0
    e = jnp.where(valid, e, 0.0)

    @pl.when(i == 0)
    def _():
        den[...] = jnp.zeros((NUM_GRAPHS, 1), jnp.float32)
        gs[...] = jnp.zeros((NUM_GRAPHS, HIDDEN), jnp.float32)

    ct = lambda a, b: lax.dot_general(a, b, (((0,), (0,)), ((), ())),
                                      preferred_element_type=jnp.float32)
    den[...] += ct(oh[...], e)
    gs[...] += ct(oh[...], e * h[...])
    gv = gs[...] / (den[...] + 1e-16)
    t = jax.nn.relu(_dot(gv, hW1[...]) + hb1[...])
    t = _ln(t, hg[...], hbn[...])
    logit[...] = _dot(t, hW2[...]) + hb2[...]


def _pool(h, gate, onehot, gmaxc, p):
    full = lambda s: pl.BlockSpec(s, lambda i: (0, 0))
    return pl.pallas_call(
        _pool_body,
        grid=(GRID,),
        in_specs=[
            pl.BlockSpec((R, HIDDEN), lambda i: (i, 0)),
            pl.BlockSpec((R, 1), lambda i: (i, 0)),
            pl.BlockSpec((R, NUM_GRAPHS), lambda i: (i, 0)),
            full((NUM_GRAPHS, 1)),
            full((HIDDEN, 64)), full((1, 64)), full((1, 64)), full((1, 64)),
            full((64, 1)), full((1, 1)),
        ],
        out_specs=[pl.BlockSpec((NUM_GRAPHS, 1), lambda i: (0, 0)),
                   pl.BlockSpec((NUM_GRAPHS, HIDDEN), lambda i: (0, 0)),
                   pl.BlockSpec((NUM_GRAPHS, 1), lambda i: (0, 0))],
        out_shape=[jax.ShapeDtypeStruct((NUM_GRAPHS, 1), jnp.float32),
                   jax.ShapeDtypeStruct((NUM_GRAPHS, HIDDEN), jnp.float32),
                   jax.ShapeDtypeStruct((NUM_GRAPHS, 1), jnp.float32)],
    )(h, gate, onehot, gmaxc,
      p['head_W1'], _r2(p['head_b1']), _r2(p['head_g']),
      _r2(p['head_bn']), p['head_W2'], _r2(p['head_b2']))


def _head_body(gs, den, W1, b1, g, bn, W2, b2, out):
    gv = gs[...] / (den[...] + 1e-16)
    t = jax.nn.relu(_dot(gv, W1[...]) + b1[...])
    t = _ln(t, g[...], bn[...])
    out[...] = _dot(t, W2[...]) + b2[...]


def _head(gs, den, p):
    full = lambda s: pl.BlockSpec(s, lambda i: (0, 0))
    return pl.pallas_call(
        _head_body,
        grid=(1,),
        in_specs=[
            full((NUM_GRAPHS, HIDDEN)), full((NUM_GRAPHS, 1)),
            full((HIDDEN, 64)), full((1, 64)), full((1, 64)), full((1, 64)),
            full((64, 1)), full((1, 1)),
        ],
        out_specs=full((NUM_GRAPHS, 1)),
        out_shape=jax.ShapeDtypeStruct((NUM_GRAPHS, 1), jnp.float32),
    )(gs, den, p['head_W1'], _r2(p['head_b1']), _r2(p['head_g']),
      _r2(p['head_bn']), p['head_W2'], _r2(p['head_b2']))


def kernel(x, params, edge_index, batch):
    p = params
    padn = lambda a: jnp.pad(a, ((0, NPAD - a.shape[0]), (0, 0)))
    x_oth = padn(jnp.pad(jnp.concatenate([x[:, :9], x[:, 15:19]], axis=1),
                         ((0, 0), (0, 3))))
    x_axis = padn(jnp.pad(x[:, 9:12], ((0, 0), (0, 5))))
    x_orig = padn(jnp.pad(x[:, 12:15], ((0, 0), (0, 5))))

    h = _stage_a(x_oth, x_axis, x_orig, p)

    bpad = jnp.pad(batch, (0, NPAD - batch.shape[0]),
                   constant_values=NUM_GRAPHS).astype(jnp.int32)
    onehot = (bpad[:, None] == jnp.arange(NUM_GRAPHS, dtype=jnp.int32)[None, :]
              ).astype(jnp.float32)
    wordrows, tbl, zeros = _edge_plan(edge_index)
    for i in range(N_LAYERS - 1):
        agg = _agg_call(h, wordrows, tbl, zeros)
        h = _layer(h, agg, p, i)
    agg = _agg_call(h, wordrows, tbl, zeros)
    h, gate, gmax = _layer_gate(h, agg, onehot, p, N_LAYERS - 1)
    den, gs, logit = _pool(h, gate, onehot, gmax.reshape(NUM_GRAPHS, 1), p)
    return logit.reshape(NUM_GRAPHS)
